# Initial kernel scaffold; baseline (speedup 1.0000x reference)
import jax
import jax.numpy as jnp
from jax.experimental import pallas as pl


def kernel(atom_number, embedding_list):
    raise NotImplementedError("write your pallas kernel here")


# speedup vs baseline: 2.2333x; 2.2333x over previous
import functools

import jax
import jax.numpy as jnp
from jax import lax
from jax.experimental import pallas as pl
from jax.experimental.pallas import tpu as pltpu
from jax.experimental.pallas import tpu_sc as plsc

N_EMB = 128
ROWS_PER_GATHER = 128
K = 4


def _emb_body(idx_hbm, table_hbm, out_hbm, idx_v, rows_v, sem):
    nc = lax.axis_size("c")
    nw = nc * lax.axis_size("s")
    wid = lax.axis_index("s") * nc + lax.axis_index("c")
    n_groups = idx_hbm.shape[0]
    per_w = n_groups // nw
    base = wid * per_w

    def step(i, carry):
        g = base + i
        pltpu.sync_copy(idx_hbm.at[g], idx_v)
        copies = [
            pltpu.async_copy(table_hbm.at[idx_v.at[j]], rows_v.at[j], sem)
            for j in range(K)
        ]
        for c in copies:
            c.wait()
        pltpu.sync_copy(rows_v, out_hbm.at[pl.ds(g * K, K)])
        return carry

    lax.fori_loop(0, per_w, step, 0)


def kernel(atom_number, embedding_list):
    b, s = atom_number.shape
    n = b * s
    n_groups = n // (K * ROWS_PER_GATHER)
    idx = atom_number.reshape(n_groups, K, ROWS_PER_GATHER).astype(jnp.int32)
    call = pl.kernel(
        _emb_body,
        out_type=jax.ShapeDtypeStruct(
            (n // ROWS_PER_GATHER, ROWS_PER_GATHER, N_EMB), jnp.float32
        ),
        mesh=plsc.VectorSubcoreMesh(core_axis_name="c", subcore_axis_name="s"),
        scratch_types=[
            pltpu.VMEM((K, ROWS_PER_GATHER), jnp.int32),
            pltpu.VMEM((K, ROWS_PER_GATHER, N_EMB), jnp.float32),
            pltpu.SemaphoreType.DMA,
        ],
    )
    out = call(idx, embedding_list.astype(jnp.float32))
    return out.reshape(b, s, N_EMB)


# speedup vs baseline: 2.2642x; 1.0138x over previous
import jax
import jax.numpy as jnp
from jax import lax
from jax.experimental import pallas as pl
from jax.experimental.pallas import tpu as pltpu
from jax.experimental.pallas import tpu_sc as plsc

N_EMB = 128
G = 128
NBUF = 3


def _emb_body(idx_hbm, table_hbm, out_hbm, idx_v, rows_v, gsem, wsem):
    nc = lax.axis_size("c")
    nw = nc * lax.axis_size("s")
    wid = lax.axis_index("s") * nc + lax.axis_index("c")
    n_groups = idx_hbm.shape[0]
    per_w = n_groups // nw
    base = wid * per_w

    pltpu.sync_copy(idx_hbm.at[pl.ds(base, per_w)], idx_v)

    def fire(p, b):
        pltpu.async_copy(table_hbm.at[idx_v.at[p]], rows_v.at[b], gsem.at[b])

    def wait_gather(b):
        pltpu.make_async_copy(out_hbm.at[0], rows_v.at[b], gsem.at[b]).wait()

    def start_wb(p, b):
        pltpu.async_copy(rows_v.at[b], out_hbm.at[base + p], wsem.at[b])

    def wait_wb(b):
        pltpu.make_async_copy(rows_v.at[b], out_hbm.at[0], wsem.at[b]).wait()

    fire(0, 0)
    fire(1, 1)

    wait_gather(0)
    start_wb(0, 0)
    fire(2, 2)
    wait_gather(1)
    start_wb(1, 1)
    wait_wb(0)
    fire(3, 0)
    wait_gather(2)
    start_wb(2, 2)
    wait_wb(1)
    fire(4, 1)

    def steady(t, carry):
        for r in range(NBUF):
            p = t * NBUF + 3 + r
            b = r
            nb = (r + 2) % NBUF
            wait_gather(b)
            start_wb(p, b)
            wait_wb(nb)
            fire(p + 2, nb)
        return carry

    lax.fori_loop(0, (per_w - 2 - 3) // NBUF, steady, 0)

    wait_gather((per_w - 2) % NBUF)
    start_wb(per_w - 2, (per_w - 2) % NBUF)
    wait_gather((per_w - 1) % NBUF)
    start_wb(per_w - 1, (per_w - 1) % NBUF)
    for b in range(NBUF):
        wait_wb(b)


def kernel(atom_number, embedding_list):
    b, s = atom_number.shape
    n = b * s
    n_groups = n // G
    idx = atom_number.reshape(n_groups, G).astype(jnp.int32)
    call = pl.kernel(
        _emb_body,
        out_type=jax.ShapeDtypeStruct((n_groups, G, N_EMB), jnp.float32),
        mesh=plsc.VectorSubcoreMesh(core_axis_name="c", subcore_axis_name="s"),
        scratch_types=[
            pltpu.VMEM((n_groups // 32, G), jnp.int32),
            pltpu.VMEM((NBUF, G, N_EMB), jnp.float32),
            pltpu.SemaphoreType.DMA((NBUF,)),
            pltpu.SemaphoreType.DMA((NBUF,)),
        ],
    )
    out = call(idx, embedding_list.astype(jnp.float32))
    return out.reshape(b, s, N_EMB)


# speedup vs baseline: 3.9662x; 1.7517x over previous
import jax
import jax.numpy as jnp
from jax import lax
from jax.experimental import pallas as pl
from jax.experimental.pallas import tpu as pltpu
from jax.experimental.pallas import tpu_sc as plsc

N_EMB = 128
G = 128
NBUF = 2
L = 16
GROUP_ELEMS = G * N_EMB


def _emb_body(idx_hbm, table_hbm, out_hbm, idx_v, table_v, rows_v, wsem):
    nc = lax.axis_size("c")
    nw = nc * lax.axis_size("s")
    wid = lax.axis_index("s") * nc + lax.axis_index("c")
    per_w = idx_hbm.shape[0] // (nw * G)
    base = wid * per_w

    pltpu.sync_copy(table_hbm, table_v)
    pltpu.sync_copy(idx_hbm.at[pl.ds(base * G, per_w * G)], idx_v)

    iota = lax.iota(jnp.int32, L)

    def bcast_lane(v, r):
        return lax.gather(
            v,
            jnp.full((L, 1), r, jnp.int32),
            lax.GatherDimensionNumbers(
                offset_dims=(), collapsed_slice_dims=(0,), start_index_map=(0,)
            ),
            (1,),
            mode=lax.GatherScatterMode.PROMISE_IN_BOUNDS,
        )

    def compute_group(p, b):
        bufbase = b * GROUP_ELEMS

        def blk_body(k, carry):
            rowpos = p * G + k * L
            idxv = idx_v[pl.ds(rowpos, L)]
            rowb = idxv * N_EMB
            for r in range(L):
                splat = bcast_lane(rowb, r)
                addr0 = splat + iota
                off = bufbase + (k * L + r) * N_EMB
                for j in range(N_EMB // L):
                    val = plsc.load_gather(table_v, [addr0 + (j * L)])
                    rows_v[pl.ds(off + j * L, L)] = val
            return carry

        lax.fori_loop(0, G // L, blk_body, 0)

    def start_wb(p, b):
        pltpu.async_copy(
            rows_v.at[pl.ds(b * GROUP_ELEMS, GROUP_ELEMS)],
            out_hbm.at[pl.ds((base + p) * GROUP_ELEMS, GROUP_ELEMS)],
            wsem.at[b],
        )

    def wait_wb(b):
        pltpu.make_async_copy(
            rows_v.at[pl.ds(b * GROUP_ELEMS, GROUP_ELEMS)],
            out_hbm.at[pl.ds(0, GROUP_ELEMS)],
            wsem.at[b],
        ).wait()

    compute_group(0, 0)
    start_wb(0, 0)
    compute_group(1, 1)
    start_wb(1, 1)

    def steady(t, carry):
        for b in range(NBUF):
            p = t * NBUF + NBUF + b
            wait_wb(b)
            compute_group(p, b)
            start_wb(p, b)
        return carry

    lax.fori_loop(0, (per_w - NBUF) // NBUF, steady, 0)

    for b in range(NBUF):
        wait_wb(b)


def kernel(atom_number, embedding_list):
    b, s = atom_number.shape
    n = b * s
    idx = atom_number.reshape(n).astype(jnp.int32)
    table = embedding_list.reshape(-1).astype(jnp.float32)
    call = pl.kernel(
        _emb_body,
        out_type=jax.ShapeDtypeStruct((n * N_EMB,), jnp.float32),
        mesh=plsc.VectorSubcoreMesh(core_axis_name="c", subcore_axis_name="s"),
        compiler_params=pltpu.CompilerParams(needs_layout_passes=False),
        scratch_types=[
            pltpu.VMEM((n // 32,), jnp.int32),
            pltpu.VMEM((table.shape[0],), jnp.float32),
            pltpu.VMEM((NBUF * GROUP_ELEMS,), jnp.float32),
            pltpu.SemaphoreType.DMA((NBUF,)),
        ],
    )
    out = call(idx, table)
    return out.reshape(b, s, N_EMB)


# speedup vs baseline: 17.5382x; 4.4219x over previous
import jax
import jax.numpy as jnp
from jax import lax
from jax.experimental import pallas as pl
from jax.experimental.pallas import tpu as pltpu
from jax.experimental.pallas import tpu_sc as plsc

N_EMB = 128
G = 128
NBUF = 2
L = 16
GROUP_ELEMS = G * N_EMB


def _emb_body(idx_hbm, table_hbm, out_hbm, idx_v, table_v, rows_v, wsem):
    nc = lax.axis_size("c")
    nw = nc * lax.axis_size("s")
    wid = lax.axis_index("s") * nc + lax.axis_index("c")
    per_w = idx_hbm.shape[0] // (nw * G)
    base = wid * per_w

    pltpu.sync_copy(table_hbm, table_v)
    pltpu.sync_copy(idx_hbm.at[pl.ds(base * G, per_w * G)], idx_v)

    iota = lax.iota(jnp.int32, L)

    def compute_group(p, b):
        bufbase = b * GROUP_ELEMS
        rowstart = p * G

        @plsc.parallel_loop(0, G, unroll=4)
        def row_body(r):
            idxsplat = plsc.load_gather(
                idx_v, [jnp.full((L,), rowstart + r, jnp.int32)]
            )
            addr0 = idxsplat * N_EMB + iota
            off = bufbase + r * N_EMB
            for j in range(N_EMB // L):
                rows_v[pl.ds(off + j * L, L)] = plsc.load_gather(
                    table_v, [addr0 + (j * L)]
                )

    def start_wb(p, b):
        pltpu.async_copy(
            rows_v.at[pl.ds(b * GROUP_ELEMS, GROUP_ELEMS)],
            out_hbm.at[pl.ds((base + p) * GROUP_ELEMS, GROUP_ELEMS)],
            wsem.at[b],
        )

    def wait_wb(b):
        pltpu.make_async_copy(
            rows_v.at[pl.ds(b * GROUP_ELEMS, GROUP_ELEMS)],
            out_hbm.at[pl.ds(0, GROUP_ELEMS)],
            wsem.at[b],
        ).wait()

    compute_group(0, 0)
    start_wb(0, 0)
    compute_group(1, 1)
    start_wb(1, 1)

    def steady(t, carry):
        for b in range(NBUF):
            p = t * NBUF + NBUF + b
            wait_wb(b)
            compute_group(p, b)
            start_wb(p, b)
        return carry

    lax.fori_loop(0, (per_w - NBUF) // NBUF, steady, 0)

    for b in range(NBUF):
        wait_wb(b)


def kernel(atom_number, embedding_list):
    b, s = atom_number.shape
    n = b * s
    idx = atom_number.reshape(n).astype(jnp.int32)
    table = embedding_list.reshape(-1).astype(jnp.float32)
    call = pl.kernel(
        _emb_body,
        out_type=jax.ShapeDtypeStruct((n * N_EMB,), jnp.float32),
        mesh=plsc.VectorSubcoreMesh(core_axis_name="c", subcore_axis_name="s"),
        compiler_params=pltpu.CompilerParams(needs_layout_passes=False),
        scratch_types=[
            pltpu.VMEM((n // 32,), jnp.int32),
            pltpu.VMEM((table.shape[0],), jnp.float32),
            pltpu.VMEM((NBUF * GROUP_ELEMS,), jnp.float32),
            pltpu.SemaphoreType.DMA((NBUF,)),
        ],
    )
    out = call(idx, table)
    return out.reshape(b, s, N_EMB)


# speedup vs baseline: 17.6234x; 1.0049x over previous
import jax
import jax.numpy as jnp
from jax import lax
from jax.experimental import pallas as pl
from jax.experimental.pallas import tpu as pltpu
from jax.experimental.pallas import tpu_sc as plsc

N_EMB = 128
G = 128
NBUF = 2
L = 16
GROUP_ELEMS = G * N_EMB


def _emb_body(idx_hbm, table_hbm, out_hbm, idx_v, table_v, rows_v, wsem):
    nc = lax.axis_size("c")
    nw = nc * lax.axis_size("s")
    wid = lax.axis_index("s") * nc + lax.axis_index("c")
    per_w = idx_hbm.shape[0] // (nw * G)
    base = wid * per_w

    pltpu.sync_copy(table_hbm, table_v)
    pltpu.sync_copy(idx_hbm.at[pl.ds(base * G, per_w * G)], idx_v)

    iota = lax.iota(jnp.int32, L)

    def compute_group(p, b):
        bufbase = b * GROUP_ELEMS
        rowstart = p * G

        @plsc.parallel_loop(0, G, unroll=8)
        def row_body(r):
            idxsplat = plsc.load_gather(
                idx_v, [jnp.full((L,), rowstart + r, jnp.int32)]
            )
            addr0 = idxsplat * N_EMB + iota
            off = bufbase + r * N_EMB
            for j in range(N_EMB // L):
                rows_v[pl.ds(off + j * L, L)] = plsc.load_gather(
                    table_v, [addr0 + (j * L)]
                )

    def start_wb(p, b):
        pltpu.async_copy(
            rows_v.at[pl.ds(b * GROUP_ELEMS, GROUP_ELEMS)],
            out_hbm.at[pl.ds((base + p) * GROUP_ELEMS, GROUP_ELEMS)],
            wsem.at[b],
        )

    def wait_wb(b):
        pltpu.make_async_copy(
            rows_v.at[pl.ds(b * GROUP_ELEMS, GROUP_ELEMS)],
            out_hbm.at[pl.ds(0, GROUP_ELEMS)],
            wsem.at[b],
        ).wait()

    compute_group(0, 0)
    start_wb(0, 0)
    compute_group(1, 1)
    start_wb(1, 1)

    def steady(t, carry):
        for b in range(NBUF):
            p = t * NBUF + NBUF + b
            wait_wb(b)
            compute_group(p, b)
            start_wb(p, b)
        return carry

    lax.fori_loop(0, (per_w - NBUF) // NBUF, steady, 0)

    for b in range(NBUF):
        wait_wb(b)


def kernel(atom_number, embedding_list):
    b, s = atom_number.shape
    n = b * s
    idx = atom_number.reshape(n).astype(jnp.int32)
    table = embedding_list.reshape(-1).astype(jnp.float32)
    call = pl.kernel(
        _emb_body,
        out_type=jax.ShapeDtypeStruct((n * N_EMB,), jnp.float32),
        mesh=plsc.VectorSubcoreMesh(core_axis_name="c", subcore_axis_name="s"),
        compiler_params=pltpu.CompilerParams(needs_layout_passes=False),
        scratch_types=[
            pltpu.VMEM((n // 32,), jnp.int32),
            pltpu.VMEM((table.shape[0],), jnp.float32),
            pltpu.VMEM((NBUF * GROUP_ELEMS,), jnp.float32),
            pltpu.SemaphoreType.DMA((NBUF,)),
        ],
    )
    out = call(idx, table)
    return out.reshape(b, s, N_EMB)


# speedup vs baseline: 17.7286x; 1.0060x over previous
import jax
import jax.numpy as jnp
from jax import lax
from jax.experimental import pallas as pl
from jax.experimental.pallas import tpu as pltpu
from jax.experimental.pallas import tpu_sc as plsc

N_EMB = 128
G = 256
NBUF = 2
L = 16
GROUP_ELEMS = G * N_EMB


def _emb_body(idx_hbm, table_hbm, out_hbm, idx_v, table_v, rows_v, wsem):
    nc = lax.axis_size("c")
    nw = nc * lax.axis_size("s")
    wid = lax.axis_index("s") * nc + lax.axis_index("c")
    per_w = idx_hbm.shape[0] // (nw * G)
    base = wid * per_w

    pltpu.sync_copy(table_hbm, table_v)
    pltpu.sync_copy(idx_hbm.at[pl.ds(base * G, per_w * G)], idx_v)

    iota = lax.iota(jnp.int32, L)

    def compute_group(p, b):
        bufbase = b * GROUP_ELEMS
        rowstart = p * G

        @plsc.parallel_loop(0, G, unroll=8)
        def row_body(r):
            idxsplat = plsc.load_gather(
                idx_v, [jnp.full((L,), rowstart + r, jnp.int32)]
            )
            addr0 = idxsplat * N_EMB + iota
            off = bufbase + r * N_EMB
            for j in range(N_EMB // L):
                rows_v[pl.ds(off + j * L, L)] = plsc.load_gather(
                    table_v, [addr0 + (j * L)]
                )

    def start_wb(p, b):
        pltpu.async_copy(
            rows_v.at[pl.ds(b * GROUP_ELEMS, GROUP_ELEMS)],
            out_hbm.at[pl.ds((base + p) * GROUP_ELEMS, GROUP_ELEMS)],
            wsem.at[b],
        )

    def wait_wb(b):
        pltpu.make_async_copy(
            rows_v.at[pl.ds(b * GROUP_ELEMS, GROUP_ELEMS)],
            out_hbm.at[pl.ds(0, GROUP_ELEMS)],
            wsem.at[b],
        ).wait()

    compute_group(0, 0)
    start_wb(0, 0)
    compute_group(1, 1)
    start_wb(1, 1)

    def steady(t, carry):
        for b in range(NBUF):
            p = t * NBUF + NBUF + b
            wait_wb(b)
            compute_group(p, b)
            start_wb(p, b)
        return carry

    lax.fori_loop(0, (per_w - NBUF) // NBUF, steady, 0)

    for b in range(NBUF):
        wait_wb(b)


def kernel(atom_number, embedding_list):
    b, s = atom_number.shape
    n = b * s
    idx = atom_number.reshape(n).astype(jnp.int32)
    table = embedding_list.reshape(-1).astype(jnp.float32)
    call = pl.kernel(
        _emb_body,
        out_type=jax.ShapeDtypeStruct((n * N_EMB,), jnp.float32),
        mesh=plsc.VectorSubcoreMesh(core_axis_name="c", subcore_axis_name="s"),
        compiler_params=pltpu.CompilerParams(needs_layout_passes=False),
        scratch_types=[
            pltpu.VMEM((n // 32,), jnp.int32),
            pltpu.VMEM((table.shape[0],), jnp.float32),
            pltpu.VMEM((NBUF * GROUP_ELEMS,), jnp.float32),
            pltpu.SemaphoreType.DMA((NBUF,)),
        ],
    )
    out = call(idx, table)
    return out.reshape(b, s, N_EMB)
